# baseline (speedup 1.0000x reference)
import jax
import jax.numpy as jnp
from jax import lax
from jax.experimental import pallas as pl
from jax.experimental.pallas import tpu as pltpu

_EPS = 1e-5


def _conv_t(xc, w_ref, KH, KW, dil, pad, W, M):
    Cin = xc.shape[0]
    PADL = pad * (W + 1)
    L = M + 2 * PADL
    xb = xc.astype(jnp.bfloat16)
    xp = jnp.pad(xb, ((0, 0), (PADL, PADL)))

    b = lax.broadcasted_iota(jnp.int32, (1, L), 1)
    wp = (b + (W - PADL % W)) % W
    zero = jnp.zeros((), jnp.bfloat16)

    masked = {}
    for kx in range(KW):
        dx = dil * kx - pad
        if dx < 0:
            masked[kx] = jnp.where(wp < W + dx, xp, zero)
        elif dx > 0:
            masked[kx] = jnp.where(wp >= dx, xp, zero)
        else:
            masked[kx] = xp

    pieces = []
    for ky in range(KH):
        for kx in range(KW):
            s = (dil * ky - pad) * W + (dil * kx - pad)
            pieces.append(lax.slice(masked[kx], (0, PADL + s),
                                    (Cin, PADL + s + M)))
    pt = jnp.concatenate(pieces, axis=0)
    return lax.dot_general(w_ref[...], pt, (((0,), (0,)), ((), ())),
                           preferred_element_type=jnp.float32)


def _make_stats_kernel(B, KH, KW, dil, pad, W, M):
    def _body(x_ref, w_ref, st_ref):
        s1 = jnp.zeros((w_ref.shape[1], 1), jnp.float32)
        s2 = s1
        for i in range(B):
            y = _conv_t(x_ref[i], w_ref, KH, KW, dil, pad, W, M)
            s1 = s1 + jnp.sum(y, axis=1, keepdims=True)
            s2 = s2 + jnp.sum(y * y, axis=1, keepdims=True)
        st_ref[0] = jnp.concatenate([s1, s2], axis=1)
    return _body


def _make_apply_kernel(B, KH, KW, dil, pad, W, M):
    def _body(x_ref, w_ref, sc_ref, sh_ref, o_ref):
        for i in range(B):
            y = _conv_t(x_ref[i], w_ref, KH, KW, dil, pad, W, M)
            o_ref[i] = y * sc_ref[...] + sh_ref[...]
    return _body


def kernel(x_nchw, w_hwio, gamma, beta):
    pad, dil = 2, 2
    N, Cin, H, W = x_nchw.shape
    KH, KW, _, Cout = w_hwio.shape
    Hout = H + 2 * pad - dil * (KH - 1)
    Wout = W + 2 * pad - dil * (KW - 1)
    assert (Hout, Wout) == (H, W), "shift-based conv assumes same-size output"
    M = H * W
    K = KH * KW * Cin
    B = 1

    x3 = x_nchw.reshape(N, Cin, M)
    w_flat = w_hwio.reshape(K, Cout).astype(jnp.bfloat16)

    stats = pl.pallas_call(
        _make_stats_kernel(B, KH, KW, dil, pad, W, M),
        out_shape=jax.ShapeDtypeStruct((N // B, Cout, 2), jnp.float32),
        grid=(N // B,),
        in_specs=[
            pl.BlockSpec((B, Cin, M), lambda n: (n, 0, 0)),
            pl.BlockSpec((K, Cout), lambda n: (0, 0)),
        ],
        out_specs=pl.BlockSpec((1, Cout, 2), lambda n: (n, 0, 0)),
        compiler_params=pltpu.CompilerParams(dimension_semantics=("parallel",)),
    )(x3, w_flat)

    cnt = jnp.float32(N * M)
    tot = jnp.sum(stats, axis=0)
    mean = tot[:, 0] / cnt
    var = jnp.maximum(tot[:, 1] / cnt - mean * mean, 0.0)
    scale = gamma.astype(jnp.float32) * lax.rsqrt(var + _EPS)
    shift = beta.astype(jnp.float32) - mean * scale

    out = pl.pallas_call(
        _make_apply_kernel(B, KH, KW, dil, pad, W, M),
        out_shape=jax.ShapeDtypeStruct((N, Cout, M), jnp.float32),
        grid=(N // B,),
        in_specs=[
            pl.BlockSpec((B, Cin, M), lambda n: (n, 0, 0)),
            pl.BlockSpec((K, Cout), lambda n: (0, 0)),
            pl.BlockSpec((Cout, 1), lambda n: (0, 0)),
            pl.BlockSpec((Cout, 1), lambda n: (0, 0)),
        ],
        out_specs=pl.BlockSpec((B, Cout, M), lambda n: (n, 0, 0)),
        compiler_params=pltpu.CompilerParams(dimension_semantics=("parallel",)),
    )(x3, w_flat, scale.reshape(Cout, 1), shift.reshape(Cout, 1))

    return out.reshape(N, Cout, Hout, Wout)


# speedup vs baseline: 1.0469x; 1.0469x over previous
import jax
import jax.numpy as jnp
from jax import lax
from jax.experimental import pallas as pl
from jax.experimental.pallas import tpu as pltpu

_EPS = 1e-5


def _conv_t(xc, w_ref, KH, KW, dil, pad, W, M):
    Cin = xc.shape[0]
    PADL = pad * (W + 1)
    L = M + 2 * PADL
    xb = xc.astype(jnp.bfloat16)
    xp = jnp.pad(xb, ((0, 0), (PADL, PADL)))

    b = lax.broadcasted_iota(jnp.int32, (1, L), 1)
    wp = (b + (W - PADL % W)) % W
    zero = jnp.zeros((), jnp.bfloat16)

    masked = {}
    for kx in range(KW):
        dx = dil * kx - pad
        if dx < 0:
            masked[kx] = jnp.where(wp < W + dx, xp, zero)
        elif dx > 0:
            masked[kx] = jnp.where(wp >= dx, xp, zero)
        else:
            masked[kx] = xp

    pieces = []
    for ky in range(KH):
        for kx in range(KW):
            s = (dil * ky - pad) * W + (dil * kx - pad)
            pieces.append(lax.slice(masked[kx], (0, PADL + s),
                                    (Cin, PADL + s + M)))
    pt = jnp.concatenate(pieces, axis=0)
    return lax.dot_general(w_ref[...], pt, (((0,), (0,)), ((), ())),
                           preferred_element_type=jnp.float32)


def _make_fused_kernel(B, S, KH, KW, dil, pad, W, M, cnt):
    def _body(x_ref, w_ref, g_ref, b_ref, o_ref, st_ref, sc_ref):
        n = pl.program_id(0)
        ys = [_conv_t(x_ref[i], w_ref, KH, KW, dil, pad, W, M)
              for i in range(B)]

        @pl.when(n == 0)
        def _init():
            st_ref[...] = jnp.zeros_like(st_ref)

        @pl.when(n < S)
        def _accum():
            a1 = jnp.zeros((st_ref.shape[0], 1), jnp.float32)
            a2 = a1
            for y in ys:
                a1 = a1 + jnp.sum(y, axis=1, keepdims=True)
                a2 = a2 + jnp.sum(y * y, axis=1, keepdims=True)
            st_ref[...] = st_ref[...] + jnp.concatenate([a1, a2], axis=1)

        @pl.when(n == S)
        def _finalize():
            tot = st_ref[...]
            mean = tot[:, 0:1] / cnt
            var = jnp.maximum(tot[:, 1:2] / cnt - mean * mean, 0.0)
            scale = g_ref[...] * lax.rsqrt(var + _EPS)
            shift = b_ref[...] - mean * scale
            sc_ref[...] = jnp.concatenate([scale, shift], axis=1)

        @pl.when(n >= S)
        def _apply():
            sc = sc_ref[...]
            scale, shift = sc[:, 0:1], sc[:, 1:2]
            for i in range(B):
                o_ref[i] = ys[i] * scale + shift
    return _body


def kernel(x_nchw, w_hwio, gamma, beta):
    pad, dil = 2, 2
    N, Cin, H, W = x_nchw.shape
    KH, KW, _, Cout = w_hwio.shape
    Hout = H + 2 * pad - dil * (KH - 1)
    Wout = W + 2 * pad - dil * (KW - 1)
    assert (Hout, Wout) == (H, W), "shift-based conv assumes same-size output"
    M = H * W
    K = KH * KW * Cin
    B = 4 if N % 4 == 0 else 1
    S = N // B

    x3 = x_nchw.reshape(N, Cin, M)
    w_flat = w_hwio.reshape(K, Cout).astype(jnp.bfloat16)
    cnt = float(N * M)

    out = pl.pallas_call(
        _make_fused_kernel(B, S, KH, KW, dil, pad, W, M, cnt),
        out_shape=jax.ShapeDtypeStruct((N, Cout, M), jnp.float32),
        grid=(2 * S,),
        in_specs=[
            pl.BlockSpec((B, Cin, M), lambda n: (n % S, 0, 0)),
            pl.BlockSpec((K, Cout), lambda n: (0, 0)),
            pl.BlockSpec((Cout, 1), lambda n: (0, 0)),
            pl.BlockSpec((Cout, 1), lambda n: (0, 0)),
        ],
        out_specs=pl.BlockSpec(
            (B, Cout, M),
            lambda n: (jnp.where(n < S, 0, n - S), 0, 0)),
        scratch_shapes=[
            pltpu.VMEM((Cout, 2), jnp.float32),
            pltpu.VMEM((Cout, 2), jnp.float32),
        ],
        compiler_params=pltpu.CompilerParams(
            dimension_semantics=("arbitrary",)),
    )(x3, w_flat, gamma.astype(jnp.float32).reshape(Cout, 1),
      beta.astype(jnp.float32).reshape(Cout, 1))

    return out.reshape(N, Cout, Hout, Wout)


# speedup vs baseline: 1.0721x; 1.0241x over previous
import jax
import jax.numpy as jnp
from jax import lax
from jax.experimental import pallas as pl
from jax.experimental.pallas import tpu as pltpu

_EPS = 1e-5


def _conv_t(xc, w_ref, KH, KW, dil, pad, W, M):
    Cin = xc.shape[0]
    PADL = pad * (W + 1)
    L = M + 2 * PADL
    xb = xc.astype(jnp.bfloat16)
    xp = jnp.pad(xb, ((0, 0), (PADL, PADL)))

    b = lax.broadcasted_iota(jnp.int32, (1, L), 1)
    wp = (b + (W - PADL % W)) % W
    zero = jnp.zeros((), jnp.bfloat16)

    masked = {}
    for kx in range(KW):
        dx = dil * kx - pad
        if dx < 0:
            masked[kx] = jnp.where(wp < W + dx, xp, zero)
        elif dx > 0:
            masked[kx] = jnp.where(wp >= dx, xp, zero)
        else:
            masked[kx] = xp

    pieces = []
    for ky in range(KH):
        for kx in range(KW):
            s = (dil * ky - pad) * W + (dil * kx - pad)
            pieces.append(lax.slice(masked[kx], (0, PADL + s),
                                    (Cin, PADL + s + M)))
    pt = jnp.concatenate(pieces, axis=0)
    return lax.dot_general(w_ref[...], pt, (((0,), (0,)), ((), ())),
                           preferred_element_type=jnp.float32)


def _make_stats_kernel(B, KH, KW, dil, pad, W, M):
    def _body(x_ref, w_ref, st_ref):
        s1 = jnp.zeros((w_ref.shape[1], 1), jnp.float32)
        s2 = s1
        for i in range(B):
            y = _conv_t(x_ref[i], w_ref, KH, KW, dil, pad, W, M)
            s1 = s1 + jnp.sum(y, axis=1, keepdims=True)
            s2 = s2 + jnp.sum(y * y, axis=1, keepdims=True)
        st_ref[0] = jnp.concatenate([s1, s2], axis=1)
    return _body


def _make_apply_kernel(B, KH, KW, dil, pad, W, M):
    def _body(x_ref, w_ref, sc_ref, sh_ref, o_ref):
        for i in range(B):
            y = _conv_t(x_ref[i], w_ref, KH, KW, dil, pad, W, M)
            o_ref[i] = y * sc_ref[...] + sh_ref[...]
    return _body


def kernel(x_nchw, w_hwio, gamma, beta):
    pad, dil = 2, 2
    N, Cin, H, W = x_nchw.shape
    KH, KW, _, Cout = w_hwio.shape
    Hout = H + 2 * pad - dil * (KH - 1)
    Wout = W + 2 * pad - dil * (KW - 1)
    assert (Hout, Wout) == (H, W), "shift-based conv assumes same-size output"
    M = H * W
    K = KH * KW * Cin
    B = 4 if N % 4 == 0 else 1

    x3 = x_nchw.reshape(N, Cin, M)
    w_flat = w_hwio.reshape(K, Cout).astype(jnp.bfloat16)

    stats = pl.pallas_call(
        _make_stats_kernel(B, KH, KW, dil, pad, W, M),
        out_shape=jax.ShapeDtypeStruct((N // B, Cout, 2), jnp.float32),
        grid=(N // B,),
        in_specs=[
            pl.BlockSpec((B, Cin, M), lambda n: (n, 0, 0)),
            pl.BlockSpec((K, Cout), lambda n: (0, 0)),
        ],
        out_specs=pl.BlockSpec((1, Cout, 2), lambda n: (n, 0, 0)),
        compiler_params=pltpu.CompilerParams(dimension_semantics=("parallel",)),
    )(x3, w_flat)

    cnt = jnp.float32(N * M)
    tot = jnp.sum(stats, axis=0)
    mean = tot[:, 0] / cnt
    var = jnp.maximum(tot[:, 1] / cnt - mean * mean, 0.0)
    scale = gamma.astype(jnp.float32) * lax.rsqrt(var + _EPS)
    shift = beta.astype(jnp.float32) - mean * scale

    out = pl.pallas_call(
        _make_apply_kernel(B, KH, KW, dil, pad, W, M),
        out_shape=jax.ShapeDtypeStruct((N, Cout, M), jnp.float32),
        grid=(N // B,),
        in_specs=[
            pl.BlockSpec((B, Cin, M), lambda n: (n, 0, 0)),
            pl.BlockSpec((K, Cout), lambda n: (0, 0)),
            pl.BlockSpec((Cout, 1), lambda n: (0, 0)),
            pl.BlockSpec((Cout, 1), lambda n: (0, 0)),
        ],
        out_specs=pl.BlockSpec((B, Cout, M), lambda n: (n, 0, 0)),
        compiler_params=pltpu.CompilerParams(dimension_semantics=("parallel",)),
    )(x3, w_flat, scale.reshape(Cout, 1), shift.reshape(Cout, 1))

    return out.reshape(N, Cout, Hout, Wout)


# speedup vs baseline: 1.0754x; 1.0030x over previous
import jax
import jax.numpy as jnp
from jax import lax
from jax.experimental import pallas as pl
from jax.experimental.pallas import tpu as pltpu

_EPS = 1e-5


def _conv_t(xc, w_ref, KH, KW, dil, pad, W, M):
    Cin = xc.shape[0]
    PADL = pad * (W + 1)
    L = M + 2 * PADL
    xb = xc.astype(jnp.bfloat16)
    xp = jnp.pad(xb, ((0, 0), (PADL, PADL)))

    b = lax.broadcasted_iota(jnp.int32, (1, L), 1)
    wp = (b + (W - PADL % W)) % W
    zero = jnp.zeros((), jnp.bfloat16)

    masked = {}
    for kx in range(KW):
        dx = dil * kx - pad
        if dx < 0:
            masked[kx] = jnp.where(wp < W + dx, xp, zero)
        elif dx > 0:
            masked[kx] = jnp.where(wp >= dx, xp, zero)
        else:
            masked[kx] = xp

    pieces = []
    for ky in range(KH):
        for kx in range(KW):
            s = (dil * ky - pad) * W + (dil * kx - pad)
            pieces.append(lax.slice(masked[kx], (0, PADL + s),
                                    (Cin, PADL + s + M)))
    pt = jnp.concatenate(pieces, axis=0)
    return lax.dot_general(w_ref[...], pt, (((0,), (0,)), ((), ())),
                           preferred_element_type=jnp.float32)


def _make_stats_kernel(B, KH, KW, dil, pad, W, M):
    def _body(x_ref, w_ref, st_ref):
        s1 = jnp.zeros((w_ref.shape[1], 1), jnp.float32)
        s2 = s1
        for i in range(B):
            y = _conv_t(x_ref[i], w_ref, KH, KW, dil, pad, W, M)
            s1 = s1 + jnp.sum(y, axis=1, keepdims=True)
            s2 = s2 + jnp.sum(y * y, axis=1, keepdims=True)
        st_ref[0] = jnp.concatenate([s1, s2], axis=1)
    return _body


def _make_apply_kernel(B, KH, KW, dil, pad, W, M):
    def _body(x_ref, w_ref, sc_ref, sh_ref, o_ref):
        for i in range(B):
            y = _conv_t(x_ref[i], w_ref, KH, KW, dil, pad, W, M)
            o_ref[i] = y * sc_ref[...] + sh_ref[...]
    return _body


def kernel(x_nchw, w_hwio, gamma, beta):
    pad, dil = 2, 2
    N, Cin, H, W = x_nchw.shape
    KH, KW, _, Cout = w_hwio.shape
    Hout = H + 2 * pad - dil * (KH - 1)
    Wout = W + 2 * pad - dil * (KW - 1)
    assert (Hout, Wout) == (H, W), "shift-based conv assumes same-size output"
    M = H * W
    K = KH * KW * Cin
    B = 8 if N % 8 == 0 else 1
    BA = 4 if N % 4 == 0 else 1

    x3 = x_nchw.reshape(N, Cin, M)
    w_flat = w_hwio.reshape(K, Cout).astype(jnp.bfloat16)

    stats = pl.pallas_call(
        _make_stats_kernel(B, KH, KW, dil, pad, W, M),
        out_shape=jax.ShapeDtypeStruct((N // B, Cout, 2), jnp.float32),
        grid=(N // B,),
        in_specs=[
            pl.BlockSpec((B, Cin, M), lambda n: (n, 0, 0)),
            pl.BlockSpec((K, Cout), lambda n: (0, 0)),
        ],
        out_specs=pl.BlockSpec((1, Cout, 2), lambda n: (n, 0, 0)),
        compiler_params=pltpu.CompilerParams(dimension_semantics=("parallel",)),
    )(x3, w_flat)

    cnt = jnp.float32(N * M)
    tot = jnp.sum(stats, axis=0)
    mean = tot[:, 0] / cnt
    var = jnp.maximum(tot[:, 1] / cnt - mean * mean, 0.0)
    scale = gamma.astype(jnp.float32) * lax.rsqrt(var + _EPS)
    shift = beta.astype(jnp.float32) - mean * scale

    out = pl.pallas_call(
        _make_apply_kernel(BA, KH, KW, dil, pad, W, M),
        out_shape=jax.ShapeDtypeStruct((N, Cout, M), jnp.float32),
        grid=(N // BA,),
        in_specs=[
            pl.BlockSpec((BA, Cin, M), lambda n: (n, 0, 0)),
            pl.BlockSpec((K, Cout), lambda n: (0, 0)),
            pl.BlockSpec((Cout, 1), lambda n: (0, 0)),
            pl.BlockSpec((Cout, 1), lambda n: (0, 0)),
        ],
        out_specs=pl.BlockSpec((BA, Cout, M), lambda n: (n, 0, 0)),
        compiler_params=pltpu.CompilerParams(dimension_semantics=("parallel",)),
    )(x3, w_flat, scale.reshape(Cout, 1), shift.reshape(Cout, 1))

    return out.reshape(N, Cout, Hout, Wout)
